# Initial kernel scaffold; baseline (speedup 1.0000x reference)
import jax
import jax.numpy as jnp
from jax.experimental import pallas as pl


def kernel(x, edge_index, W1, b1, W2, b2, W3, b3, W4, b4, W5, b5):
    raise NotImplementedError("write your pallas kernel here")


# speedup vs baseline: 24.3589x; 24.3589x over previous
import jax
import jax.numpy as jnp
from jax import lax
from jax.experimental import pallas as pl
from jax.experimental.pallas import tpu as pltpu
from jax.experimental.pallas import tpu_sc as plsc

N = 10000
E = 320000
D = 128
DO = 32

NC = 2
NS = 16
NW = NC * NS
EW = E // NW
CH = 80
NCH = EW // CH
IG = 25
NPG = NCH // IG
NP = 10240
RPT = NP // NS
WBR = 64
NWB = RPT // WBR

_mesh = plsc.VectorSubcoreMesh(core_axis_name="c", subcore_axis_name="s")


def _deg_body(dst_hbm, out_hbm, dstv, onesv, zer, idxv, degsh, sem):
    c = lax.axis_index("c")
    s = lax.axis_index("s")
    w = c * NS + s
    ii = lax.iota(jnp.int32, 16)

    for r in range(8):
        for q in range(5):
            idxv[r, pl.ds(q * 16, 16)] = s * RPT + (r * 80 + q * 16) + ii

    def _fill(r, _):
        for col in range(D // 16):
            onesv[r, pl.ds(col * 16, 16)] = jnp.ones((16,), jnp.float32)
            zer[r, pl.ds(col * 16, 16)] = jnp.zeros((16,), jnp.float32)
        return 0
    lax.fori_loop(0, CH, _fill, 0)

    for r in range(8):
        pltpu.sync_copy(zer, degsh.at[idxv.at[r]])
    pltpu.sync_copy(dst_hbm.at[w], dstv)
    plsc.subcore_barrier()

    for i in range(NCH):
        pltpu.sync_copy(onesv, degsh.at[dstv.at[i]], add=True)
    plsc.subcore_barrier()

    for r in range(8):
        pltpu.sync_copy(degsh.at[idxv.at[r]], zer)
        pltpu.sync_copy(zer, out_hbm.at[c, pl.ds(s * RPT + r * 80, 80)])


_deg_call = pl.kernel(
    _deg_body,
    out_type=jax.ShapeDtypeStruct((NC, NP, D), jnp.float32),
    mesh=_mesh,
    scratch_types=[
        pltpu.VMEM((NCH, CH), jnp.int32),
        pltpu.VMEM((CH, D), jnp.float32),
        pltpu.VMEM((CH, D), jnp.float32),
        pltpu.VMEM((8, CH), jnp.int32),
        pltpu.VMEM_SHARED((NP, D), jnp.float32),
        pltpu.SemaphoreType.DMA,
    ],
)


def _seg_body(xs_hbm, src_hbm, dst_hbm, out_hbm,
              srcr, dstr, bufa, bufb, idxv, accsh,
              gsa, gsb, pss, psd):
    c = lax.axis_index("c")
    s = lax.axis_index("s")
    w = c * NS + s
    ii = lax.iota(jnp.int32, 16)

    for r in range(8):
        for q in range(5):
            idxv[r, pl.ds(q * 16, 16)] = s * RPT + (r * 80 + q * 16) + ii

    def _zrow(r, _):
        for col in range(D // 16):
            bufa[r, pl.ds(col * 16, 16)] = jnp.zeros((16,), jnp.float32)
        return 0
    lax.fori_loop(0, CH, _zrow, 0)
    for r in range(8):
        pltpu.sync_copy(bufa, accsh.at[idxv.at[r]])

    pltpu.sync_copy(src_hbm.at[w, 0], srcr.at[0])
    pltpu.sync_copy(dst_hbm.at[w, 0], dstr.at[0])
    plsc.subcore_barrier()

    bufs = (bufa, bufb)
    gsems = (gsa, gsb)

    pend = {}
    pend[0] = pltpu.async_copy(xs_hbm.at[srcr.at[0, 0]], bufs[0], gsems[0])
    ppend = []
    for i in range(NCH):
        pg, k = divmod(i, IG)
        par = pg % 2
        if k == 0 and pg + 1 < NPG:
            npar = (pg + 1) % 2
            ppend = [
                pltpu.async_copy(src_hbm.at[w, pg + 1], srcr.at[npar], pss),
                pltpu.async_copy(dst_hbm.at[w, pg + 1], dstr.at[npar], psd),
            ]
        if i + 1 < NCH:
            npg, nk = divmod(i + 1, IG)
            if nk == 0:
                for h in ppend:
                    h.wait()
                ppend = []
            nb = (i + 1) % 2
            pend[nb] = pltpu.async_copy(
                xs_hbm.at[srcr.at[npg % 2, nk]], bufs[nb], gsems[nb])
        b = i % 2
        pend[b].wait()
        pltpu.sync_copy(bufs[b], accsh.at[dstr.at[par, k]], add=True)
    plsc.subcore_barrier()

    for r in range(8):
        pltpu.sync_copy(accsh.at[idxv.at[r]], bufa)
        pltpu.sync_copy(bufa, out_hbm.at[c, pl.ds(s * RPT + r * 80, 80)])


_seg_call = pl.kernel(
    _seg_body,
    out_type=jax.ShapeDtypeStruct((NC, NP, D), jnp.float32),
    mesh=_mesh,
    scratch_types=[
        pltpu.VMEM((2, IG, CH), jnp.int32),
        pltpu.VMEM((2, IG, CH), jnp.int32),
        pltpu.VMEM((CH, D), jnp.float32),
        pltpu.VMEM((CH, D), jnp.float32),
        pltpu.VMEM((8, CH), jnp.int32),
        pltpu.VMEM_SHARED((NP, D), jnp.float32),
        pltpu.SemaphoreType.DMA,
        pltpu.SemaphoreType.DMA,
        pltpu.SemaphoreType.DMA,
        pltpu.SemaphoreType.DMA,
    ],
)


BN = 1000
GRID = N // BN

_b2d = lambda i: (i, 0)
_b3d = lambda i: (0, i, 0)
_w2d = lambda i: (0, 0)


def _k0_body(x_ref, w_ref, dp_ref, xs_ref, dis_ref):
    deg = dp_ref[0, :, 0:1] + dp_ref[1, :, 0:1] + 1.0
    dis = lax.rsqrt(deg)
    xw = jnp.dot(x_ref[:], w_ref[:], preferred_element_type=jnp.float32)
    xs_ref[:] = xw * dis
    dis_ref[:] = dis


_k0_call = pl.pallas_call(
    _k0_body,
    grid=(GRID,),
    in_specs=[
        pl.BlockSpec((BN, D), _b2d),
        pl.BlockSpec((D, D), _w2d),
        pl.BlockSpec((NC, BN, D), _b3d),
    ],
    out_specs=[
        pl.BlockSpec((BN, D), _b2d),
        pl.BlockSpec((BN, 1), _b2d),
    ],
    out_shape=[
        jax.ShapeDtypeStruct((N, D), jnp.float32),
        jax.ShapeDtypeStruct((N, 1), jnp.float32),
    ],
)


def _mid_body(p_ref, xs_ref, dis_ref, b_ref, w_ref, out_ref):
    dis = dis_ref[:]
    conv = dis * (p_ref[0] + p_ref[1] + xs_ref[:]) + b_ref[:]
    h = jnp.maximum(conv, 0.0)
    out_ref[:] = jnp.dot(h, w_ref[:], preferred_element_type=jnp.float32) * dis


_mid_call = pl.pallas_call(
    _mid_body,
    grid=(GRID,),
    in_specs=[
        pl.BlockSpec((NC, BN, D), _b3d),
        pl.BlockSpec((BN, D), _b2d),
        pl.BlockSpec((BN, 1), _b2d),
        pl.BlockSpec((1, D), _w2d),
        pl.BlockSpec((D, D), _w2d),
    ],
    out_specs=pl.BlockSpec((BN, D), _b2d),
    out_shape=jax.ShapeDtypeStruct((N, D), jnp.float32),
)


def _head_body(p_ref, xs_ref, dis_ref, b3_ref, w4_ref, b4_ref,
               w5_ref, b5_ref, emb_ref, out_ref):
    dis = dis_ref[:]
    emb = dis * (p_ref[0] + p_ref[1] + xs_ref[:]) + b3_ref[:]
    emb_ref[:] = emb
    h = jnp.maximum(emb, 0.0)
    m = jnp.dot(h, w4_ref[:], preferred_element_type=jnp.float32) + b4_ref[:]
    m = jnp.maximum(m, 0.0)
    logits = jnp.dot(m, w5_ref[:], preferred_element_type=jnp.float32) + b5_ref[:]
    mx = jnp.max(logits, axis=1, keepdims=True)
    ex = jnp.exp(logits - mx)
    lse = jnp.log(jnp.sum(ex, axis=1, keepdims=True)) + mx
    out_ref[:] = logits - lse


_head_call = pl.pallas_call(
    _head_body,
    grid=(GRID,),
    in_specs=[
        pl.BlockSpec((NC, BN, D), _b3d),
        pl.BlockSpec((BN, D), _b2d),
        pl.BlockSpec((BN, 1), _b2d),
        pl.BlockSpec((1, D), _w2d),
        pl.BlockSpec((D, D), _w2d),
        pl.BlockSpec((1, D), _w2d),
        pl.BlockSpec((D, DO), _w2d),
        pl.BlockSpec((1, DO), _w2d),
    ],
    out_specs=[
        pl.BlockSpec((BN, D), _b2d),
        pl.BlockSpec((BN, DO), _b2d),
    ],
    out_shape=[
        jax.ShapeDtypeStruct((N, D), jnp.float32),
        jax.ShapeDtypeStruct((N, DO), jnp.float32),
    ],
)


def kernel(x, edge_index, W1, b1, W2, b2, W3, b3, W4, b4, W5, b5):
    src_p = edge_index[0].reshape(NW, NPG, IG, CH)
    dst_p = edge_index[1].reshape(NW, NPG, IG, CH)
    dst_w = edge_index[1].reshape(NW, NCH, CH)

    degp = _deg_call(dst_w)
    xs, dis = _k0_call(x, W1, degp)

    p = _seg_call(xs, src_p, dst_p)
    xs = _mid_call(p, xs, dis, b1.reshape(1, D), W2)

    p = _seg_call(xs, src_p, dst_p)
    xs = _mid_call(p, xs, dis, b2.reshape(1, D), W3)

    p = _seg_call(xs, src_p, dst_p)
    emb, out = _head_call(p, xs, dis, b3.reshape(1, D),
                          W4, b4.reshape(1, D), W5, b5.reshape(1, DO))
    return (emb, out)


# speedup vs baseline: 26.9892x; 1.1080x over previous
import jax
import jax.numpy as jnp
from jax import lax
from jax.experimental import pallas as pl
from jax.experimental.pallas import tpu as pltpu
from jax.experimental.pallas import tpu_sc as plsc

N = 10000
E = 320000
D = 128
DO = 32

NC = 2
NS = 16
NW = NC * NS
EW = E // NW
CH = 80
NCH = EW // CH
IG = 5
NPG = NCH // IG
NP = 10240
RPT = NP // NS
WBR = 64
NWB = RPT // WBR

_mesh = plsc.VectorSubcoreMesh(core_axis_name="c", subcore_axis_name="s")


def _deg_body(dst_hbm, out_hbm, dstv, onesv, zer, idxv, degsh, sem):
    c = lax.axis_index("c")
    s = lax.axis_index("s")
    w = c * NS + s
    ii = lax.iota(jnp.int32, 16)

    for r in range(8):
        for q in range(5):
            idxv[r, pl.ds(q * 16, 16)] = s * RPT + (r * 80 + q * 16) + ii

    def _fill(r, _):
        for col in range(D // 16):
            onesv[r, pl.ds(col * 16, 16)] = jnp.ones((16,), jnp.float32)
            zer[r, pl.ds(col * 16, 16)] = jnp.zeros((16,), jnp.float32)
        return 0
    lax.fori_loop(0, CH, _fill, 0)

    for r in range(8):
        pltpu.sync_copy(zer, degsh.at[idxv.at[r]])
    pltpu.sync_copy(dst_hbm.at[w], dstv)
    plsc.subcore_barrier()

    WAVE = 5
    for g in range(NCH // WAVE):
        handles = [
            pltpu.async_copy(onesv, degsh.at[dstv.at[i]], sem, add=True)
            for i in range(g * WAVE, (g + 1) * WAVE)]
        for h in handles:
            h.wait()
    plsc.subcore_barrier()

    for r in range(8):
        pltpu.sync_copy(degsh.at[idxv.at[r]], zer)
        pltpu.sync_copy(zer, out_hbm.at[c, pl.ds(s * RPT + r * 80, 80)])


_deg_call = pl.kernel(
    _deg_body,
    out_type=jax.ShapeDtypeStruct((NC, NP, D), jnp.float32),
    mesh=_mesh,
    scratch_types=[
        pltpu.VMEM((NCH, CH), jnp.int32),
        pltpu.VMEM((CH, D), jnp.float32),
        pltpu.VMEM((CH, D), jnp.float32),
        pltpu.VMEM((8, CH), jnp.int32),
        pltpu.VMEM_SHARED((NP, D), jnp.float32),
        pltpu.SemaphoreType.DMA,
    ],
)


def _seg_body(xs_hbm, src_hbm, dst_hbm, out_hbm,
              srcr, dstr, b0, b1, b2, b3, idxv, accsh,
              g0, g1, g2, g3, s0, s1, s2, s3, pss, psd):
    c = lax.axis_index("c")
    s = lax.axis_index("s")
    w = c * NS + s
    ii = lax.iota(jnp.int32, 16)

    for r in range(8):
        for q in range(5):
            idxv[r, pl.ds(q * 16, 16)] = s * RPT + (r * 80 + q * 16) + ii

    def _zrow(r, _):
        for col in range(D // 16):
            b0[r, pl.ds(col * 16, 16)] = jnp.zeros((16,), jnp.float32)
        return 0
    lax.fori_loop(0, CH, _zrow, 0)
    for r in range(8):
        pltpu.sync_copy(b0, accsh.at[idxv.at[r]])

    pltpu.sync_copy(src_hbm.at[w, 0], srcr.at[0])
    pltpu.sync_copy(dst_hbm.at[w, 0], dstr.at[0])
    plsc.subcore_barrier()

    bufs = (b0, b1, b2, b3)
    gsems = (g0, g1, g2, g3)
    ssems = (s0, s1, s2, s3)

    pend_g = {}
    pend_s = {}
    ppend = []
    pend_g[0] = pltpu.async_copy(xs_hbm.at[srcr.at[0, 0]], bufs[0], gsems[0])
    pend_g[1] = pltpu.async_copy(xs_hbm.at[srcr.at[0, 1]], bufs[1], gsems[1])
    for j in range(NCH):
        pg, k = divmod(j, IG)
        if k == 0 and pg + 1 < NPG:
            npar = (pg + 1) % 2
            ppend = [
                pltpu.async_copy(src_hbm.at[w, pg + 1], srcr.at[npar], pss),
                pltpu.async_copy(dst_hbm.at[w, pg + 1], dstr.at[npar], psd),
            ]
        if j >= 2:
            pend_s[(j - 2) % 4].wait()
        if j + 2 < NCH:
            npg, nk = divmod(j + 2, IG)
            if nk == 0:
                for h in ppend:
                    h.wait()
                ppend = []
            nb = (j + 2) % 4
            pend_g[nb] = pltpu.async_copy(
                xs_hbm.at[srcr.at[npg % 2, nk]], bufs[nb], gsems[nb])
        b = j % 4
        pend_g[b].wait()
        pend_s[b] = pltpu.async_copy(
            bufs[b], accsh.at[dstr.at[pg % 2, k]], ssems[b], add=True)
    pend_s[(NCH - 2) % 4].wait()
    pend_s[(NCH - 1) % 4].wait()
    plsc.subcore_barrier()

    for r in range(8):
        pltpu.sync_copy(accsh.at[idxv.at[r]], b0)
        pltpu.sync_copy(b0, out_hbm.at[c, pl.ds(s * RPT + r * 80, 80)])


_seg_call = pl.kernel(
    _seg_body,
    out_type=jax.ShapeDtypeStruct((NC, NP, D), jnp.float32),
    mesh=_mesh,
    scratch_types=[
        pltpu.VMEM((2, IG, CH), jnp.int32),
        pltpu.VMEM((2, IG, CH), jnp.int32),
        pltpu.VMEM((CH, D), jnp.float32),
        pltpu.VMEM((CH, D), jnp.float32),
        pltpu.VMEM((CH, D), jnp.float32),
        pltpu.VMEM((CH, D), jnp.float32),
        pltpu.VMEM((8, CH), jnp.int32),
        pltpu.VMEM_SHARED((NP, D), jnp.float32),
        pltpu.SemaphoreType.DMA,
        pltpu.SemaphoreType.DMA,
        pltpu.SemaphoreType.DMA,
        pltpu.SemaphoreType.DMA,
        pltpu.SemaphoreType.DMA,
        pltpu.SemaphoreType.DMA,
        pltpu.SemaphoreType.DMA,
        pltpu.SemaphoreType.DMA,
        pltpu.SemaphoreType.DMA,
        pltpu.SemaphoreType.DMA,
    ],
)


BN = 1000
GRID = N // BN

_b2d = lambda i: (i, 0)
_b3d = lambda i: (0, i, 0)
_w2d = lambda i: (0, 0)


def _k0_body(x_ref, w_ref, dp_ref, xs_ref, dis_ref):
    deg = dp_ref[0, :, 0:1] + dp_ref[1, :, 0:1] + 1.0
    dis = lax.rsqrt(deg)
    xw = jnp.dot(x_ref[:], w_ref[:], preferred_element_type=jnp.float32)
    xs_ref[:] = xw * dis
    dis_ref[:] = dis


_k0_call = pl.pallas_call(
    _k0_body,
    grid=(GRID,),
    in_specs=[
        pl.BlockSpec((BN, D), _b2d),
        pl.BlockSpec((D, D), _w2d),
        pl.BlockSpec((NC, BN, D), _b3d),
    ],
    out_specs=[
        pl.BlockSpec((BN, D), _b2d),
        pl.BlockSpec((BN, 1), _b2d),
    ],
    out_shape=[
        jax.ShapeDtypeStruct((N, D), jnp.float32),
        jax.ShapeDtypeStruct((N, 1), jnp.float32),
    ],
)


def _mid_body(p_ref, xs_ref, dis_ref, b_ref, w_ref, out_ref):
    dis = dis_ref[:]
    conv = dis * (p_ref[0] + p_ref[1] + xs_ref[:]) + b_ref[:]
    h = jnp.maximum(conv, 0.0)
    out_ref[:] = jnp.dot(h, w_ref[:], preferred_element_type=jnp.float32) * dis


_mid_call = pl.pallas_call(
    _mid_body,
    grid=(GRID,),
    in_specs=[
        pl.BlockSpec((NC, BN, D), _b3d),
        pl.BlockSpec((BN, D), _b2d),
        pl.BlockSpec((BN, 1), _b2d),
        pl.BlockSpec((1, D), _w2d),
        pl.BlockSpec((D, D), _w2d),
    ],
    out_specs=pl.BlockSpec((BN, D), _b2d),
    out_shape=jax.ShapeDtypeStruct((N, D), jnp.float32),
)


def _head_body(p_ref, xs_ref, dis_ref, b3_ref, w4_ref, b4_ref,
               w5_ref, b5_ref, emb_ref, out_ref):
    dis = dis_ref[:]
    emb = dis * (p_ref[0] + p_ref[1] + xs_ref[:]) + b3_ref[:]
    emb_ref[:] = emb
    h = jnp.maximum(emb, 0.0)
    m = jnp.dot(h, w4_ref[:], preferred_element_type=jnp.float32) + b4_ref[:]
    m = jnp.maximum(m, 0.0)
    logits = jnp.dot(m, w5_ref[:], preferred_element_type=jnp.float32) + b5_ref[:]
    mx = jnp.max(logits, axis=1, keepdims=True)
    ex = jnp.exp(logits - mx)
    lse = jnp.log(jnp.sum(ex, axis=1, keepdims=True)) + mx
    out_ref[:] = logits - lse


_head_call = pl.pallas_call(
    _head_body,
    grid=(GRID,),
    in_specs=[
        pl.BlockSpec((NC, BN, D), _b3d),
        pl.BlockSpec((BN, D), _b2d),
        pl.BlockSpec((BN, 1), _b2d),
        pl.BlockSpec((1, D), _w2d),
        pl.BlockSpec((D, D), _w2d),
        pl.BlockSpec((1, D), _w2d),
        pl.BlockSpec((D, DO), _w2d),
        pl.BlockSpec((1, DO), _w2d),
    ],
    out_specs=[
        pl.BlockSpec((BN, D), _b2d),
        pl.BlockSpec((BN, DO), _b2d),
    ],
    out_shape=[
        jax.ShapeDtypeStruct((N, D), jnp.float32),
        jax.ShapeDtypeStruct((N, DO), jnp.float32),
    ],
)


def kernel(x, edge_index, W1, b1, W2, b2, W3, b3, W4, b4, W5, b5):
    src_p = edge_index[0].reshape(NW, NPG, IG, CH)
    dst_p = edge_index[1].reshape(NW, NPG, IG, CH)
    dst_w = edge_index[1].reshape(NW, NCH, CH)

    degp = _deg_call(dst_w)
    xs, dis = _k0_call(x, W1, degp)

    p = _seg_call(xs, src_p, dst_p)
    xs = _mid_call(p, xs, dis, b1.reshape(1, D), W2)

    p = _seg_call(xs, src_p, dst_p)
    xs = _mid_call(p, xs, dis, b2.reshape(1, D), W3)

    p = _seg_call(xs, src_p, dst_p)
    emb, out = _head_call(p, xs, dis, b3.reshape(1, D),
                          W4, b4.reshape(1, D), W5, b5.reshape(1, DO))
    return (emb, out)


# speedup vs baseline: 27.0293x; 1.0015x over previous
import jax
import jax.numpy as jnp
from jax import lax
from jax.experimental import pallas as pl
from jax.experimental.pallas import tpu as pltpu
from jax.experimental.pallas import tpu_sc as plsc

N = 10000
E = 320000
D = 128
DO = 32

NC = 2
NS = 16
NW = NC * NS
EW = E // NW
CH = 80
NCH = EW // CH
IG = 5
NPG = NCH // IG
NP = 10240
RPT = NP // NS
WBR = 64
NWB = RPT // WBR

_mesh = plsc.VectorSubcoreMesh(core_axis_name="c", subcore_axis_name="s")


def _deg_body(dst_hbm, out_hbm, dstv, onesv, zer, idxv, degsh, sem):
    c = lax.axis_index("c")
    s = lax.axis_index("s")
    w = c * NS + s
    ii = lax.iota(jnp.int32, 16)

    for r in range(8):
        for q in range(5):
            idxv[r, pl.ds(q * 16, 16)] = s * RPT + (r * 80 + q * 16) + ii

    def _fill(r, _):
        for col in range(D // 16):
            onesv[r, pl.ds(col * 16, 16)] = jnp.ones((16,), jnp.float32)
            zer[r, pl.ds(col * 16, 16)] = jnp.zeros((16,), jnp.float32)
        return 0
    lax.fori_loop(0, CH, _fill, 0)

    for r in range(8):
        pltpu.sync_copy(zer, degsh.at[idxv.at[r]])
    pltpu.sync_copy(dst_hbm.at[w], dstv)
    plsc.subcore_barrier()

    WAVE = 5
    for g in range(NCH // WAVE):
        handles = [
            pltpu.async_copy(onesv, degsh.at[dstv.at[i]], sem, add=True)
            for i in range(g * WAVE, (g + 1) * WAVE)]
        for h in handles:
            h.wait()
    plsc.subcore_barrier()

    for r in range(8):
        pltpu.sync_copy(degsh.at[idxv.at[r]], zer)
        pltpu.sync_copy(zer, out_hbm.at[c, pl.ds(s * RPT + r * 80, 80)])


_deg_call = pl.kernel(
    _deg_body,
    out_type=jax.ShapeDtypeStruct((NC, NP, D), jnp.float32),
    mesh=_mesh,
    scratch_types=[
        pltpu.VMEM((NCH, CH), jnp.int32),
        pltpu.VMEM((CH, D), jnp.float32),
        pltpu.VMEM((CH, D), jnp.float32),
        pltpu.VMEM((8, CH), jnp.int32),
        pltpu.VMEM_SHARED((NP, D), jnp.float32),
        pltpu.SemaphoreType.DMA,
    ],
)


def _seg_body(xs_hbm, src_hbm, dst_hbm, out_hbm,
              srcr, dstr, b0, b1, b2, b3, idxv, accsh,
              g0, g1, g2, g3, s0, s1, s2, s3, pss, psd):
    c = lax.axis_index("c")
    s = lax.axis_index("s")
    w = c * NS + s
    ii = lax.iota(jnp.int32, 16)

    for r in range(8):
        for q in range(5):
            idxv[r, pl.ds(q * 16, 16)] = s * RPT + (r * 80 + q * 16) + ii

    def _zrow(r, _):
        for col in range(D // 16):
            b0[r, pl.ds(col * 16, 16)] = jnp.zeros((16,), jnp.float32)
        return 0
    lax.fori_loop(0, CH, _zrow, 0)
    for r in range(8):
        pltpu.sync_copy(b0, accsh.at[idxv.at[r]])

    pltpu.sync_copy(src_hbm.at[w, 0], srcr.at[0])
    pltpu.sync_copy(dst_hbm.at[w, 0], dstr.at[0])
    plsc.subcore_barrier()

    bufs = (b0, b1, b2, b3)
    gsems = (g0, g1, g2, g3)
    ssems = (s0, s1, s2, s3)

    pend_g = {}
    pend_s = {}
    ppend = []
    pend_g[0] = pltpu.async_copy(xs_hbm.at[srcr.at[0, 0]], bufs[0], gsems[0])
    pend_g[1] = pltpu.async_copy(xs_hbm.at[srcr.at[0, 1]], bufs[1], gsems[1])
    for j in range(NCH):
        pg, k = divmod(j, IG)
        if k == 0 and pg + 1 < NPG:
            npar = (pg + 1) % 2
            ppend = [
                pltpu.async_copy(src_hbm.at[w, pg + 1], srcr.at[npar], pss),
                pltpu.async_copy(dst_hbm.at[w, pg + 1], dstr.at[npar], psd),
            ]
        if j >= 2:
            pend_s[(j - 2) % 4].wait()
        if j + 2 < NCH:
            npg, nk = divmod(j + 2, IG)
            if nk == 0:
                for h in ppend:
                    h.wait()
                ppend = []
            nb = (j + 2) % 4
            pend_g[nb] = pltpu.async_copy(
                xs_hbm.at[srcr.at[npg % 2, nk]], bufs[nb], gsems[nb])
        b = j % 4
        pend_g[b].wait()
        pend_s[b] = pltpu.async_copy(
            bufs[b], accsh.at[dstr.at[pg % 2, k]], ssems[b], add=True)
    pend_s[(NCH - 2) % 4].wait()
    pend_s[(NCH - 1) % 4].wait()
    plsc.subcore_barrier()

    for r in range(8):
        pltpu.sync_copy(accsh.at[idxv.at[r]], b0)
        pltpu.sync_copy(b0, out_hbm.at[c, pl.ds(s * RPT + r * 80, 80)])


_seg_call = pl.kernel(
    _seg_body,
    out_type=jax.ShapeDtypeStruct((NC, NP, D), jnp.float32),
    mesh=_mesh,
    scratch_types=[
        pltpu.VMEM((2, IG, CH), jnp.int32),
        pltpu.VMEM((2, IG, CH), jnp.int32),
        pltpu.VMEM((CH, D), jnp.float32),
        pltpu.VMEM((CH, D), jnp.float32),
        pltpu.VMEM((CH, D), jnp.float32),
        pltpu.VMEM((CH, D), jnp.float32),
        pltpu.VMEM((8, CH), jnp.int32),
        pltpu.VMEM_SHARED((NP, D), jnp.float32),
        pltpu.SemaphoreType.DMA,
        pltpu.SemaphoreType.DMA,
        pltpu.SemaphoreType.DMA,
        pltpu.SemaphoreType.DMA,
        pltpu.SemaphoreType.DMA,
        pltpu.SemaphoreType.DMA,
        pltpu.SemaphoreType.DMA,
        pltpu.SemaphoreType.DMA,
        pltpu.SemaphoreType.DMA,
        pltpu.SemaphoreType.DMA,
    ],
)


BN = 1000
GRID = N // BN

_b2d = lambda i: (i, 0)
_b3d = lambda i: (0, i, 0)
_w2d = lambda i: (0, 0)


def _k0a_body(x_ref, w_ref, xw_ref):
    xw_ref[:] = jnp.dot(x_ref[:], w_ref[:], preferred_element_type=jnp.float32)


_k0a_call = pl.pallas_call(
    _k0a_body,
    grid=(GRID,),
    in_specs=[
        pl.BlockSpec((BN, D), _b2d),
        pl.BlockSpec((D, D), _w2d),
    ],
    out_specs=pl.BlockSpec((BN, D), _b2d),
    out_shape=jax.ShapeDtypeStruct((N, D), jnp.float32),
)


def _k0b_body(xw_ref, dp_ref, xs_ref, dis_ref):
    deg = dp_ref[0, :, 0:1] + dp_ref[1, :, 0:1] + 1.0
    dis = lax.rsqrt(deg)
    xs_ref[:] = xw_ref[:] * dis
    dis_ref[:] = dis


_k0b_call = pl.pallas_call(
    _k0b_body,
    grid=(GRID,),
    in_specs=[
        pl.BlockSpec((BN, D), _b2d),
        pl.BlockSpec((NC, BN, D), _b3d),
    ],
    out_specs=[
        pl.BlockSpec((BN, D), _b2d),
        pl.BlockSpec((BN, 1), _b2d),
    ],
    out_shape=[
        jax.ShapeDtypeStruct((N, D), jnp.float32),
        jax.ShapeDtypeStruct((N, 1), jnp.float32),
    ],
)


def _mid_body(p_ref, xs_ref, dis_ref, b_ref, w_ref, out_ref):
    dis = dis_ref[:]
    conv = dis * (p_ref[0] + p_ref[1] + xs_ref[:]) + b_ref[:]
    h = jnp.maximum(conv, 0.0)
    out_ref[:] = jnp.dot(h, w_ref[:], preferred_element_type=jnp.float32) * dis


_mid_call = pl.pallas_call(
    _mid_body,
    grid=(GRID,),
    in_specs=[
        pl.BlockSpec((NC, BN, D), _b3d),
        pl.BlockSpec((BN, D), _b2d),
        pl.BlockSpec((BN, 1), _b2d),
        pl.BlockSpec((1, D), _w2d),
        pl.BlockSpec((D, D), _w2d),
    ],
    out_specs=pl.BlockSpec((BN, D), _b2d),
    out_shape=jax.ShapeDtypeStruct((N, D), jnp.float32),
)


def _head_body(p_ref, xs_ref, dis_ref, b3_ref, w4_ref, b4_ref,
               w5_ref, b5_ref, emb_ref, out_ref):
    dis = dis_ref[:]
    emb = dis * (p_ref[0] + p_ref[1] + xs_ref[:]) + b3_ref[:]
    emb_ref[:] = emb
    h = jnp.maximum(emb, 0.0)
    m = jnp.dot(h, w4_ref[:], preferred_element_type=jnp.float32) + b4_ref[:]
    m = jnp.maximum(m, 0.0)
    logits = jnp.dot(m, w5_ref[:], preferred_element_type=jnp.float32) + b5_ref[:]
    mx = jnp.max(logits, axis=1, keepdims=True)
    ex = jnp.exp(logits - mx)
    lse = jnp.log(jnp.sum(ex, axis=1, keepdims=True)) + mx
    out_ref[:] = logits - lse


_head_call = pl.pallas_call(
    _head_body,
    grid=(GRID,),
    in_specs=[
        pl.BlockSpec((NC, BN, D), _b3d),
        pl.BlockSpec((BN, D), _b2d),
        pl.BlockSpec((BN, 1), _b2d),
        pl.BlockSpec((1, D), _w2d),
        pl.BlockSpec((D, D), _w2d),
        pl.BlockSpec((1, D), _w2d),
        pl.BlockSpec((D, DO), _w2d),
        pl.BlockSpec((1, DO), _w2d),
    ],
    out_specs=[
        pl.BlockSpec((BN, D), _b2d),
        pl.BlockSpec((BN, DO), _b2d),
    ],
    out_shape=[
        jax.ShapeDtypeStruct((N, D), jnp.float32),
        jax.ShapeDtypeStruct((N, DO), jnp.float32),
    ],
)


def kernel(x, edge_index, W1, b1, W2, b2, W3, b3, W4, b4, W5, b5):
    src_p = edge_index[0].reshape(NW, NPG, IG, CH)
    dst_p = edge_index[1].reshape(NW, NPG, IG, CH)
    dst_w = edge_index[1].reshape(NW, NCH, CH)

    xw1 = _k0a_call(x, W1)
    degp = _deg_call(dst_w)
    xs, dis = _k0b_call(xw1, degp)

    p = _seg_call(xs, src_p, dst_p)
    xs = _mid_call(p, xs, dis, b1.reshape(1, D), W2)

    p = _seg_call(xs, src_p, dst_p)
    xs = _mid_call(p, xs, dis, b2.reshape(1, D), W3)

    p = _seg_call(xs, src_p, dst_p)
    emb, out = _head_call(p, xs, dis, b3.reshape(1, D),
                          W4, b4.reshape(1, D), W5, b5.reshape(1, DO))
    return (emb, out)
